# Initial kernel scaffold; baseline (speedup 1.0000x reference)
import jax
import jax.numpy as jnp
from jax.experimental import pallas as pl


def kernel(inputs, w_gate, w0_kernel, w1_kernel, wo_kernel):
    raise NotImplementedError("write your pallas kernel here")


# speedup vs baseline: 2.4010x; 2.4010x over previous
import functools

import jax
import jax.numpy as jnp
from jax.experimental import pallas as pl
from jax.experimental.pallas import tpu as pltpu

_NUM_EXPERTS = 8
_TOP_K = 2
_EMB = 1024
_MLP = 4096
_ROWS = 4096
_TILE = 512
_NT = _ROWS // _TILE
_STEPS = _NT + _NUM_EXPERTS - 1
_MC = 1024
_NK = _MLP // _MC


def _gmm_body(tile_ref, expert_ref, lo_ref, hi_ref,
              x_ref, w0_ref, w1_ref, wo_ref, out_ref):
    s = pl.program_id(0)
    k = pl.program_id(1)
    lo = lo_ref[s]
    hi = hi_ref[s]
    rows = jax.lax.broadcasted_iota(jnp.int32, (_TILE, 1), 0)
    mask = (rows >= lo) & (rows < hi)
    x = jnp.where(mask, x_ref[...], jnp.zeros_like(x_ref[...]))
    a0 = jax.lax.dot_general(x, w0_ref[0], (((1,), (0,)), ((), ())),
                             preferred_element_type=jnp.float32)
    a1 = jax.lax.dot_general(x, w1_ref[0], (((1,), (0,)), ((), ())),
                             preferred_element_type=jnp.float32)
    act = (a0 * jax.nn.sigmoid(a0) * a1).astype(x.dtype)
    contrib = jax.lax.dot_general(act, wo_ref[0], (((1,), (0,)), ((), ())),
                                  preferred_element_type=jnp.float32)

    prev_tile = tile_ref[jnp.maximum(s - 1, 0)]
    first_visit = jnp.logical_and(
        k == 0, jnp.logical_or(s == 0, tile_ref[s] != prev_tile))

    @pl.when(first_visit)
    def _():
        out_ref[...] = contrib

    @pl.when(jnp.logical_not(first_visit))
    def _():
        out_ref[...] += contrib


def _gmm(sorted_x, w0, w1, wo, step_tile, step_expert, step_lo, step_hi):
    grid_spec = pltpu.PrefetchScalarGridSpec(
        num_scalar_prefetch=4,
        grid=(_STEPS, _NK),
        in_specs=[
            pl.BlockSpec((_TILE, _EMB), lambda s, k, t, e, lo, hi: (t[s], 0)),
            pl.BlockSpec((1, _EMB, _MC), lambda s, k, t, e, lo, hi: (e[s], 0, k)),
            pl.BlockSpec((1, _EMB, _MC), lambda s, k, t, e, lo, hi: (e[s], 0, k)),
            pl.BlockSpec((1, _MC, _EMB), lambda s, k, t, e, lo, hi: (e[s], k, 0)),
        ],
        out_specs=pl.BlockSpec((_TILE, _EMB), lambda s, k, t, e, lo, hi: (t[s], 0)),
    )
    return pl.pallas_call(
        _gmm_body,
        grid_spec=grid_spec,
        out_shape=jax.ShapeDtypeStruct((_ROWS, _EMB), jnp.float32),
    )(step_tile, step_expert, step_lo, step_hi, sorted_x, w0, w1, wo)


def _schedule(group_sizes):
    offs = jnp.concatenate([jnp.zeros((1,), jnp.int32),
                            jnp.cumsum(group_sizes).astype(jnp.int32)])
    t = jnp.arange(_NT, dtype=jnp.int32)[:, None]
    start = jnp.maximum(offs[:-1][None, :], t * _TILE)
    end = jnp.minimum(offs[1:][None, :], (t + 1) * _TILE)
    active = (end > start).ravel()
    slot = jnp.where(active, jnp.cumsum(active.astype(jnp.int32)) - 1, _STEPS)
    tiles = jnp.broadcast_to(t, (_NT, _NUM_EXPERTS)).ravel()
    experts = jnp.broadcast_to(jnp.arange(_NUM_EXPERTS, dtype=jnp.int32)[None, :],
                               (_NT, _NUM_EXPERTS)).ravel()
    lo = (start - t * _TILE).ravel()
    hi = (end - t * _TILE).ravel()

    def fill(vals, default):
        buf = jnp.full((_STEPS + 1,), default, jnp.int32)
        return buf.at[slot].set(vals.astype(jnp.int32), mode="drop")[:_STEPS]

    step_tile = fill(tiles, _NT - 1)
    step_expert = fill(experts, _NUM_EXPERTS - 1)
    step_lo = fill(lo, 0)
    step_hi = fill(hi, 0)
    return step_tile, step_expert, step_lo, step_hi


@jax.jit
def kernel(inputs, w_gate, w0_kernel, w1_kernel, wo_kernel):
    x2d = inputs.reshape(-1, _EMB)
    logits = x2d @ w_gate
    weights, selected = jax.lax.top_k(logits, _TOP_K)
    weights = jax.nn.softmax(weights.astype(jnp.float32), axis=-1)
    flat = selected.ravel()
    sort_idx = jnp.argsort(flat)
    sorted_x = jnp.take(x2d, sort_idx // _TOP_K, axis=0)
    group_sizes = jnp.bincount(flat, length=_NUM_EXPERTS)

    sched = _schedule(group_sizes)
    inter = _gmm(sorted_x.astype(jnp.bfloat16),
                 w0_kernel.astype(jnp.bfloat16),
                 w1_kernel.astype(jnp.bfloat16),
                 wo_kernel.astype(jnp.bfloat16), *sched)

    unsorted = jnp.take(inter, jnp.argsort(sort_idx), axis=0)
    out = jnp.einsum("tke,tk->te", unsorted.reshape(-1, _TOP_K, _EMB), weights)
    return out.reshape(inputs.shape)


# speedup vs baseline: 2.4677x; 1.0278x over previous
import functools

import jax
import jax.numpy as jnp
from jax.experimental import pallas as pl
from jax.experimental.pallas import tpu as pltpu

_NUM_EXPERTS = 8
_TOP_K = 2
_EMB = 1024
_MLP = 4096
_ROWS = 4096
_TILE = 512
_NT = _ROWS // _TILE
_STEPS = _NT + _NUM_EXPERTS - 1
_MC = 1024
_NK = _MLP // _MC


def _gmm_body(tile_ref, expert_ref, lo_ref, hi_ref,
              x_ref, w0_ref, w1_ref, wo_ref, out_ref):
    s = pl.program_id(0)
    lo = lo_ref[s]
    hi = hi_ref[s]
    rows = jax.lax.broadcasted_iota(jnp.int32, (_TILE, 1), 0)
    mask = (rows >= lo) & (rows < hi)
    x = jnp.where(mask, x_ref[...], jnp.zeros_like(x_ref[...]))
    a0 = jax.lax.dot_general(x, w0_ref[0], (((1,), (0,)), ((), ())),
                             preferred_element_type=jnp.float32)
    a1 = jax.lax.dot_general(x, w1_ref[0], (((1,), (0,)), ((), ())),
                             preferred_element_type=jnp.float32)
    act = (a0 * jax.nn.sigmoid(a0) * a1).astype(x.dtype)
    contrib = jax.lax.dot_general(act, wo_ref[0], (((1,), (0,)), ((), ())),
                                  preferred_element_type=jnp.float32)

    prev_tile = tile_ref[jnp.maximum(s - 1, 0)]
    first_visit = jnp.logical_or(s == 0, tile_ref[s] != prev_tile)

    @pl.when(first_visit)
    def _():
        out_ref[...] = contrib

    @pl.when(jnp.logical_not(first_visit))
    def _():
        out_ref[...] += contrib


def _gmm(sorted_x, w0, w1, wo, step_tile, step_expert, step_lo, step_hi):
    grid_spec = pltpu.PrefetchScalarGridSpec(
        num_scalar_prefetch=4,
        grid=(_STEPS,),
        in_specs=[
            pl.BlockSpec((_TILE, _EMB), lambda s, t, e, lo, hi: (t[s], 0)),
            pl.BlockSpec((1, _EMB, _MLP), lambda s, t, e, lo, hi: (e[s], 0, 0)),
            pl.BlockSpec((1, _EMB, _MLP), lambda s, t, e, lo, hi: (e[s], 0, 0)),
            pl.BlockSpec((1, _MLP, _EMB), lambda s, t, e, lo, hi: (e[s], 0, 0)),
        ],
        out_specs=pl.BlockSpec((_TILE, _EMB), lambda s, t, e, lo, hi: (t[s], 0)),
    )
    return pl.pallas_call(
        _gmm_body,
        grid_spec=grid_spec,
        out_shape=jax.ShapeDtypeStruct((_ROWS, _EMB), jnp.float32),
        compiler_params=pltpu.CompilerParams(
            vmem_limit_bytes=110 * 1024 * 1024),
    )(step_tile, step_expert, step_lo, step_hi, sorted_x, w0, w1, wo)


def _schedule(group_sizes):
    offs = jnp.concatenate([jnp.zeros((1,), jnp.int32),
                            jnp.cumsum(group_sizes).astype(jnp.int32)])
    t = jnp.arange(_NT, dtype=jnp.int32)[:, None]
    start = jnp.maximum(offs[:-1][None, :], t * _TILE)
    end = jnp.minimum(offs[1:][None, :], (t + 1) * _TILE)
    active = (end > start).ravel()
    slot = jnp.where(active, jnp.cumsum(active.astype(jnp.int32)) - 1, _STEPS)
    tiles = jnp.broadcast_to(t, (_NT, _NUM_EXPERTS)).ravel()
    experts = jnp.broadcast_to(jnp.arange(_NUM_EXPERTS, dtype=jnp.int32)[None, :],
                               (_NT, _NUM_EXPERTS)).ravel()
    lo = (start - t * _TILE).ravel()
    hi = (end - t * _TILE).ravel()

    def fill(vals, default):
        buf = jnp.full((_STEPS + 1,), default, jnp.int32)
        return buf.at[slot].set(vals.astype(jnp.int32), mode="drop")[:_STEPS]

    step_tile = fill(tiles, _NT - 1)
    step_expert = fill(experts, _NUM_EXPERTS - 1)
    step_lo = fill(lo, 0)
    step_hi = fill(hi, 0)
    return step_tile, step_expert, step_lo, step_hi


@jax.jit
def kernel(inputs, w_gate, w0_kernel, w1_kernel, wo_kernel):
    x2d = inputs.reshape(-1, _EMB)
    logits = x2d @ w_gate
    weights, selected = jax.lax.top_k(logits, _TOP_K)
    weights = jax.nn.softmax(weights.astype(jnp.float32), axis=-1)
    flat = selected.ravel()
    sort_idx = jnp.argsort(flat)
    sorted_x = jnp.take(x2d, sort_idx // _TOP_K, axis=0)
    group_sizes = jnp.bincount(flat, length=_NUM_EXPERTS)

    sched = _schedule(group_sizes)
    inter = _gmm(sorted_x.astype(jnp.bfloat16),
                 w0_kernel.astype(jnp.bfloat16),
                 w1_kernel.astype(jnp.bfloat16),
                 wo_kernel.astype(jnp.bfloat16), *sched)

    unsorted = jnp.take(inter, jnp.argsort(sort_idx), axis=0)
    out = jnp.einsum("tke,tk->te", unsorted.reshape(-1, _TOP_K, _EMB), weights)
    return out.reshape(inputs.shape)


# speedup vs baseline: 2.8703x; 1.1632x over previous
import functools

import jax
import jax.numpy as jnp
from jax.experimental import pallas as pl
from jax.experimental.pallas import tpu as pltpu

_NUM_EXPERTS = 8
_TOP_K = 2
_EMB = 1024
_MLP = 4096
_ROWS = 4096
_TILE = 256
_NT = _ROWS // _TILE
_STEPS = _NT + _NUM_EXPERTS - 1


def _k1_body(tile_ref, expert_ref, lo_ref, hi_ref,
             x_ref, w0_ref, w1_ref, act_ref, w0b_ref, w1b_ref):
    s = pl.program_id(1)
    new_expert = jnp.logical_or(
        s == 0, expert_ref[s] != expert_ref[jnp.maximum(s - 1, 0)])

    @pl.when(new_expert)
    def _():
        w0b_ref[...] = w0_ref[0].astype(jnp.bfloat16)
        w1b_ref[...] = w1_ref[0].astype(jnp.bfloat16)

    x = x_ref[...]
    a0 = jax.lax.dot_general(x, w0b_ref[...], (((1,), (0,)), ((), ())),
                             preferred_element_type=jnp.float32)
    a1 = jax.lax.dot_general(x, w1b_ref[...], (((1,), (0,)), ((), ())),
                             preferred_element_type=jnp.float32)
    act_ref[...] = (a0 * jax.nn.sigmoid(a0) * a1).astype(jnp.bfloat16)


def _k2_body(tile_ref, expert_ref, lo_ref, hi_ref,
             act_ref, wo_ref, out_ref, wob_ref):
    s = pl.program_id(0)
    new_expert = jnp.logical_or(
        s == 0, expert_ref[s] != expert_ref[jnp.maximum(s - 1, 0)])

    @pl.when(new_expert)
    def _():
        wob_ref[...] = wo_ref[0].astype(jnp.bfloat16)

    contrib = jax.lax.dot_general(act_ref[...], wob_ref[...],
                                  (((1,), (0,)), ((), ())),
                                  preferred_element_type=jnp.float32)
    rows = jax.lax.broadcasted_iota(jnp.int32, (_TILE, 1), 0)
    mask = (rows >= lo_ref[s]) & (rows < hi_ref[s])
    contrib = jnp.where(mask, contrib, jnp.zeros_like(contrib))

    prev_tile = tile_ref[jnp.maximum(s - 1, 0)]
    first_visit = jnp.logical_or(s == 0, tile_ref[s] != prev_tile)

    @pl.when(first_visit)
    def _():
        out_ref[...] = contrib

    @pl.when(jnp.logical_not(first_visit))
    def _():
        out_ref[...] += contrib


_MC = 1024
_NK = _MLP // _MC


def _gmm(sorted_x, w0, w1, wo, step_tile, step_expert, step_lo, step_hi):
    sched = (step_tile, step_expert, step_lo, step_hi)
    k1_spec = pltpu.PrefetchScalarGridSpec(
        num_scalar_prefetch=4,
        grid=(_NK, _STEPS),
        in_specs=[
            pl.BlockSpec((_TILE, _EMB), lambda k, s, t, e, lo, hi: (t[s], 0)),
            pl.BlockSpec((1, _EMB, _MC), lambda k, s, t, e, lo, hi: (e[s], 0, k)),
            pl.BlockSpec((1, _EMB, _MC), lambda k, s, t, e, lo, hi: (e[s], 0, k)),
        ],
        out_specs=pl.BlockSpec((_TILE, _MC), lambda k, s, t, e, lo, hi: (s, k)),
        scratch_shapes=[
            pltpu.VMEM((_EMB, _MC), jnp.bfloat16),
            pltpu.VMEM((_EMB, _MC), jnp.bfloat16),
        ],
    )
    act = pl.pallas_call(
        _k1_body,
        grid_spec=k1_spec,
        out_shape=jax.ShapeDtypeStruct((_STEPS * _TILE, _MLP), jnp.bfloat16),
        compiler_params=pltpu.CompilerParams(
            vmem_limit_bytes=60 * 1024 * 1024),
    )(*sched, sorted_x, w0, w1)

    k2_spec = pltpu.PrefetchScalarGridSpec(
        num_scalar_prefetch=4,
        grid=(_STEPS,),
        in_specs=[
            pl.BlockSpec((_TILE, _MLP), lambda s, t, e, lo, hi: (s, 0)),
            pl.BlockSpec((1, _MLP, _EMB), lambda s, t, e, lo, hi: (e[s], 0, 0)),
        ],
        out_specs=pl.BlockSpec((_TILE, _EMB), lambda s, t, e, lo, hi: (t[s], 0)),
        scratch_shapes=[pltpu.VMEM((_MLP, _EMB), jnp.bfloat16)],
    )
    return pl.pallas_call(
        _k2_body,
        grid_spec=k2_spec,
        out_shape=jax.ShapeDtypeStruct((_ROWS, _EMB), jnp.float32),
        compiler_params=pltpu.CompilerParams(
            vmem_limit_bytes=60 * 1024 * 1024),
    )(*sched, act, wo)


def _schedule(group_sizes):
    offs = jnp.concatenate([jnp.zeros((1,), jnp.int32),
                            jnp.cumsum(group_sizes).astype(jnp.int32)])
    t = jnp.arange(_NT, dtype=jnp.int32)[:, None]
    start = jnp.maximum(offs[:-1][None, :], t * _TILE)
    end = jnp.minimum(offs[1:][None, :], (t + 1) * _TILE)
    active = (end > start).ravel()
    slot = jnp.where(active, jnp.cumsum(active.astype(jnp.int32)) - 1, _STEPS)
    tiles = jnp.broadcast_to(t, (_NT, _NUM_EXPERTS)).ravel()
    experts = jnp.broadcast_to(jnp.arange(_NUM_EXPERTS, dtype=jnp.int32)[None, :],
                               (_NT, _NUM_EXPERTS)).ravel()
    lo = (start - t * _TILE).ravel()
    hi = (end - t * _TILE).ravel()

    def fill(vals, default):
        buf = jnp.full((_STEPS + 1,), default, jnp.int32)
        return buf.at[slot].set(vals.astype(jnp.int32), mode="drop")[:_STEPS]

    step_tile = fill(tiles, _NT - 1)
    step_expert = fill(experts, _NUM_EXPERTS - 1)
    step_lo = fill(lo, 0)
    step_hi = fill(hi, 0)
    return step_tile, step_expert, step_lo, step_hi


@jax.jit
def kernel(inputs, w_gate, w0_kernel, w1_kernel, wo_kernel):
    x2d = inputs.reshape(-1, _EMB)
    logits = x2d @ w_gate
    weights, selected = jax.lax.top_k(logits, _TOP_K)
    weights = jax.nn.softmax(weights.astype(jnp.float32), axis=-1)
    flat = selected.ravel()
    sort_idx = jnp.argsort(flat)
    sorted_x = jnp.take(x2d, sort_idx // _TOP_K, axis=0)
    group_sizes = jnp.bincount(flat, length=_NUM_EXPERTS)

    sched = _schedule(group_sizes)
    inter = _gmm(sorted_x.astype(jnp.bfloat16),
                 w0_kernel, w1_kernel, wo_kernel, *sched)

    unsorted = jnp.take(inter, jnp.argsort(sort_idx), axis=0)
    out = jnp.einsum("tke,tk->te", unsorted.reshape(-1, _TOP_K, _EMB), weights)
    return out.reshape(inputs.shape)
